# baseline (speedup 1.0000x reference)
import functools

import jax
import jax.numpy as jnp
from jax import lax
from jax.experimental import pallas as pl
from jax.experimental.pallas import tpu as pltpu
import jax.experimental.pallas.tpu_sc as plsc

_NC = 2
_NS = 16
_NW = _NC * _NS
_C = 128


def _scale_body(h_ref, norm_ref, o_ref):
    o_ref[...] = h_ref[...] * norm_ref[...]


_B = 2
_WV = _B * _C


def _make_sc_body(w_per, rps, n_pad):
    n_chunks = w_per // _C
    n_waves = n_chunks // _B

    def body(norm_h_hbm, src_hbm, dst2_hbm, z_hbm, out_hbm,
             src_v, dst_v2, rows_v, acc_sh, gsems, ssems, isems):
        c = lax.axis_index("c")
        s = lax.axis_index("s")
        wid = s * _NC + c
        row0 = pl.multiple_of(s * rps, 8)
        ebase = pl.multiple_of(wid * w_per, 8)
        cbase = pl.multiple_of(wid * n_chunks, 8)

        def src_load(gi, slot, sem):
            return pltpu.async_copy(
                src_hbm.at[pl.ds(pl.multiple_of(ebase + gi * _WV, 8), _WV)],
                src_v.at[slot], sem)

        def gather(gi, b, slot, sem):
            off = pl.multiple_of(b * _C, 8)
            return pltpu.async_copy(
                norm_h_hbm.at[src_v.at[slot].at[pl.ds(off, _C)]],
                rows_v.at[b], sem)

        pltpu.sync_copy(z_hbm, acc_sh.at[pl.ds(row0, rps)])
        pltpu.sync_copy(dst2_hbm.at[pl.ds(cbase, n_chunks)], dst_v2)
        src_load(0, 0, isems[0]).wait()
        if n_waves > 1:
            src_load(1, 1, isems[1])
        plsc.subcore_barrier()
        for b in range(_B):
            gather(0, b, 0, gsems[b])

        def wave(gi, slot):
            sdescs = []
            for b in range(_B):
                pltpu.make_async_copy(
                    norm_h_hbm.at[src_v.at[slot].at[pl.ds(b * _C, _C)]],
                    rows_v.at[b], gsems[b]).wait()
                j = gi * _B + b
                sdescs.append(pltpu.async_copy(
                    rows_v.at[b], acc_sh.at[dst_v2.at[j]], ssems[b], add=True))
            for b in range(_B):
                sdescs[b].wait()

            @pl.when(gi + 2 < n_waves)
            def _():
                src_load(gi + 2, slot, isems[slot])

            @pl.when(gi + 1 < n_waves)
            def _():
                pltpu.make_async_copy(
                    src_hbm.at[pl.ds(ebase, _WV)],
                    src_v.at[1 - slot], isems[1 - slot]).wait()
                for b in range(_B):
                    gather(gi + 1, b, 1 - slot, gsems[b])

        def pair(g2, carry):
            wave(g2 * 2, 0)
            wave(g2 * 2 + 1, 1)
            return carry

        lax.fori_loop(0, n_waves // 2, pair, 0)
        plsc.subcore_barrier()
        out0 = pl.multiple_of(c * n_pad + s * rps, 8)
        pltpu.sync_copy(acc_sh.at[pl.ds(row0, rps)], out_hbm.at[pl.ds(out0, rps)])

    return body


def _mm_body(h_ref, p_ref, norm_ref, w_ref, w2_ref, o_ref):
    nrm = norm_ref[...]
    agg = (p_ref[0] + p_ref[1]) * nrm
    x = jnp.concatenate([h_ref[...], agg], axis=1)
    y = jnp.dot(x, w_ref[...], preferred_element_type=jnp.float32)
    y = jnp.maximum(y, 0.0)
    ss = jnp.sum(y * y, axis=1, keepdims=True)
    y = y * lax.rsqrt(jnp.maximum(ss, 1e-12))
    o_ref[...] = jnp.maximum(
        jnp.dot(y, w2_ref[...], preferred_element_type=jnp.float32), 0.0)


def kernel(h, edge_index, norm, weight, weight2):
    n, d = h.shape
    e = edge_index.shape[1]
    d_out = weight2.shape[1]

    src = edge_index[0].astype(jnp.int32)
    dst = edge_index[1].astype(jnp.int32)

    w_per = -(-e // (_NW * _WV * 2)) * (_WV * 2)
    e_pad = w_per * _NW
    n_pad = -(-n // _C) * _C
    rps = n_pad // _NS
    if e_pad > e:
        src = jnp.concatenate([src, jnp.zeros((e_pad - e,), jnp.int32)])
        dst = jnp.concatenate([dst, jnp.full((e_pad - e,), n, jnp.int32)])

    bn = 1000 if n % 1000 == 0 else n

    norm_h = pl.pallas_call(
        _scale_body,
        out_shape=jax.ShapeDtypeStruct((n, d), jnp.float32),
        grid=(n // bn,),
        in_specs=[pl.BlockSpec((bn, d), lambda i: (i, 0)),
                  pl.BlockSpec((bn, 1), lambda i: (i, 0))],
        out_specs=pl.BlockSpec((bn, d), lambda i: (i, 0)),
    )(h, norm)

    z = jnp.zeros((rps, d), jnp.float32)
    mesh = plsc.VectorSubcoreMesh(core_axis_name="c", subcore_axis_name="s")
    dst2 = dst.reshape(e_pad // _C, _C)
    partial = pl.kernel(
        _make_sc_body(w_per, rps, n_pad),
        out_type=jax.ShapeDtypeStruct((_NC * n_pad, d), jnp.float32),
        mesh=mesh,
        scratch_types=[
            pltpu.VMEM((2, _WV), jnp.int32),
            pltpu.VMEM((w_per // _C, _C), jnp.int32),
            pltpu.VMEM((_B, _C, d), jnp.float32),
            pltpu.VMEM_SHARED((n_pad, d), jnp.float32),
            [pltpu.SemaphoreType.DMA] * _B,
            [pltpu.SemaphoreType.DMA] * _B,
            [pltpu.SemaphoreType.DMA] * 2,
        ],
    )(norm_h, src, dst2, z)
    p = partial.reshape(_NC, n_pad, d)

    out = pl.pallas_call(
        _mm_body,
        out_shape=jax.ShapeDtypeStruct((n, d_out), jnp.float32),
        grid=(n // bn,),
        in_specs=[
            pl.BlockSpec((bn, d), lambda i: (i, 0)),
            pl.BlockSpec((_NC, bn, d), lambda i: (0, i, 0)),
            pl.BlockSpec((bn, 1), lambda i: (i, 0)),
            pl.BlockSpec(weight.shape, lambda i: (0, 0)),
            pl.BlockSpec(weight2.shape, lambda i: (0, 0)),
        ],
        out_specs=pl.BlockSpec((bn, d_out), lambda i: (i, 0)),
    )(h, p, norm, weight, weight2)
    return out
